# Initial kernel scaffold; baseline (speedup 1.0000x reference)
import jax
import jax.numpy as jnp
from jax.experimental import pallas as pl


def kernel(h, e, edge_index0, edge_index1, W, b):
    raise NotImplementedError("write your pallas kernel here")


# speedup vs baseline: 2.0554x; 2.0554x over previous
import jax
import jax.numpy as jnp
from jax import lax
from jax.experimental import pallas as pl
from jax.experimental.pallas import tpu as pltpu
from jax.experimental.pallas import tpu_sc as plsc

N_NODES = 10000
N_EDGES = 320000
D = 128
H = 64

NC = 2
NS = 16

EB = 128
NBLK = N_EDGES // EB
NP = 10240
NROW = NP // NS
RCH = 128


def _node_mm_body(h_ref, w_ref, b_ref, ah_ref, t_ref):
    p = jnp.dot(h_ref[...], w_ref[...], preferred_element_type=jnp.float32)
    p = p + b_ref[...]
    ah_ref[...] = p[:, 0:D]
    for c in range(NC):
        for t, col0 in ((0, D), (1, 2 * D), (2, 3 * D)):
            lo = col0 + c * H
            t_ref[c * 3 + t] = p[:, lo:lo + H]


def _node_mm(h, wcat, bcat):
    bn = 1000
    grid = (N_NODES // bn,)
    return pl.pallas_call(
        _node_mm_body,
        grid=grid,
        in_specs=[
            pl.BlockSpec((bn, D), lambda i: (i, 0)),
            pl.BlockSpec((D, 4 * D), lambda i: (0, 0)),
            pl.BlockSpec((1, 4 * D), lambda i: (0, 0)),
        ],
        out_specs=[
            pl.BlockSpec((bn, D), lambda i: (i, 0)),
            pl.BlockSpec((2 * 3, bn, H), lambda i: (0, i, 0)),
        ],
        out_shape=[
            jax.ShapeDtypeStruct((N_NODES, D), jnp.float32),
            jax.ShapeDtypeStruct((2 * 3, N_NODES, H), jnp.float32),
        ],
    )(h, wcat, bcat)


def _edge_mm_dense_body(e_ref, w_ref, b_ref, ce_ref):
    p = jnp.dot(e_ref[...], w_ref[...], preferred_element_type=jnp.float32)
    p = p + b_ref[...]
    ce_ref[0] = p[:, 0:H]
    ce_ref[1] = p[:, H:D]


def _edge_mm_split_body(e_ref, w_ref, b_ref, ce_ref):
    ecat = jnp.concatenate([e_ref[0], e_ref[1]], axis=1)
    p = jnp.dot(ecat, w_ref[...], preferred_element_type=jnp.float32)
    p = p + b_ref[...]
    ce_ref[0] = p[:, 0:H]
    ce_ref[1] = p[:, H:D]


def _edge_mm(e, wc, bc):
    be = 2000
    grid = (N_EDGES // be,)
    if e.ndim == 2:
        body = _edge_mm_dense_body
        espec = pl.BlockSpec((be, D), lambda i: (i, 0))
    else:
        body = _edge_mm_split_body
        espec = pl.BlockSpec((NC, be, H), lambda i: (0, i, 0))
    return pl.pallas_call(
        body,
        grid=grid,
        in_specs=[
            espec,
            pl.BlockSpec((D, D), lambda i: (0, 0)),
            pl.BlockSpec((1, D), lambda i: (0, 0)),
        ],
        out_specs=[pl.BlockSpec((NC, be, H), lambda i: (0, i, 0))],
        out_shape=[jax.ShapeDtypeStruct((NC, N_EDGES, H), jnp.float32)],
    )(e, wc, bc)[0]


def _node_upd_body(ah_ref, num_ref, den_ref, h_ref):
    num = jnp.concatenate([num_ref[0], num_ref[1]], axis=1)
    den = jnp.concatenate([den_ref[0], den_ref[1]], axis=1)
    h_ref[...] = jnp.maximum(ah_ref[...] + num / (den + 1e-6), 0.0)


def _node_upd(ah, num, den):
    bn = 2000
    grid = (N_NODES // bn,)
    spec = pl.BlockSpec((bn, D), lambda i: (i, 0))
    hspec = pl.BlockSpec((NC, bn, H), lambda i: (0, i, 0))
    return pl.pallas_call(
        _node_upd_body,
        grid=grid,
        in_specs=[spec, hspec, hspec],
        out_specs=[spec],
        out_shape=[jax.ShapeDtypeStruct((N_NODES, D), jnp.float32)],
    )(ah, num, den)[0]


def _merge_body(e_ref, out_ref):
    out_ref[...] = jnp.concatenate([e_ref[0], e_ref[1]], axis=1)


def _merge_e(e2):
    be = 4000
    grid = (N_EDGES // be,)
    return pl.pallas_call(
        _merge_body,
        grid=grid,
        in_specs=[pl.BlockSpec((NC, be, H), lambda i: (0, i, 0))],
        out_specs=[pl.BlockSpec((be, D), lambda i: (i, 0))],
        out_shape=[jax.ShapeDtypeStruct((N_EDGES, D), jnp.float32)],
    )(e2)[0]


def _edge_sc_body(t_hbm, ce_hbm, src_hbm, dst_hbm,
                  eout_hbm, num_hbm, den_hbm,
                  src_v, dst_v, ib_v, id_v, ie_v,
                  rb, rd, re, cev, stage,
                  num_sh, den_sh, gsem):
    c = lax.axis_index("c")
    s = lax.axis_index("s")
    base = c * (3 * N_NODES)

    zero16 = jnp.zeros((16,), jnp.float32)

    def _zrow(r, _):
        for k in range(H // 16):
            stage[r, pl.ds(k * 16, 16)] = zero16
        return 0

    lax.fori_loop(0, RCH, _zrow, 0)
    r0 = s * NROW
    for j in range(NROW // RCH):
        pltpu.sync_copy(stage, num_sh.at[pl.ds(r0 + j * RCH, RCH)])
        pltpu.sync_copy(stage, den_sh.at[pl.ds(r0 + j * RCH, RCH)])
    plsc.subcore_barrier()

    nblk = (NBLK - s + NS - 1) // NS

    def _blk(g, _):
        off = (s + g * NS) * EB
        pltpu.sync_copy(src_hbm.at[pl.ds(off, EB)], src_v)
        pltpu.sync_copy(dst_hbm.at[pl.ds(off, EB)], dst_v)
        for j in range(EB // 16):
            sl = pl.ds(j * 16, 16)
            sv = src_v[sl]
            dv = dst_v[sl]
            ib_v[sl] = sv + base
            id_v[sl] = sv + (base + N_NODES)
            ie_v[sl] = dv + (base + 2 * N_NODES)
        cp1 = pltpu.async_copy(t_hbm.at[ib_v], rb, gsem)
        cp2 = pltpu.async_copy(t_hbm.at[id_v], rd, gsem)
        cp3 = pltpu.async_copy(t_hbm.at[ie_v], re, gsem)
        pltpu.sync_copy(ce_hbm.at[pl.ds(c * N_EDGES + off, EB)], cev)
        cp1.wait()
        cp2.wait()
        cp3.wait()

        def _row(r, _):
            for k in range(H // 16):
                sl2 = pl.ds(k * 16, 16)
                en = rd[r, sl2] + re[r, sl2] + cev[r, sl2]
                sg = 1.0 / (1.0 + jnp.exp(-en))
                rd[r, sl2] = jnp.maximum(en, 0.0)
                rb[r, sl2] = sg * rb[r, sl2]
                re[r, sl2] = sg
            return 0

        lax.fori_loop(0, EB, _row, 0)

        pltpu.sync_copy(rd, eout_hbm.at[pl.ds(c * N_EDGES + off, EB)])
        pltpu.sync_copy(rb, num_sh.at[dst_v], add=True)
        pltpu.sync_copy(re, den_sh.at[dst_v], add=True)
        return 0

    lax.fori_loop(0, nblk, _blk, 0)
    plsc.subcore_barrier()

    for j in range(NROW // RCH):
        rr = pl.ds(r0 + j * RCH, RCH)
        pltpu.sync_copy(num_sh.at[rr], stage)
        pltpu.sync_copy(stage, num_hbm.at[pl.ds(c * NP + r0 + j * RCH, RCH)])
        pltpu.sync_copy(den_sh.at[rr], stage)
        pltpu.sync_copy(stage, den_hbm.at[pl.ds(c * NP + r0 + j * RCH, RCH)])


_sc_mesh = plsc.VectorSubcoreMesh(
    core_axis_name="c", subcore_axis_name="s", num_cores=NC, num_subcores=NS)

_edge_sc = pl.kernel(
    _edge_sc_body,
    out_type=[
        jax.ShapeDtypeStruct((NC * N_EDGES, H), jnp.float32),
        jax.ShapeDtypeStruct((NC * NP, H), jnp.float32),
        jax.ShapeDtypeStruct((NC * NP, H), jnp.float32),
    ],
    mesh=_sc_mesh,
    compiler_params=pltpu.CompilerParams(use_tc_tiling_on_sc=False),
    scratch_types=[
        pltpu.VMEM((EB,), jnp.int32),
        pltpu.VMEM((EB,), jnp.int32),
        pltpu.VMEM((EB,), jnp.int32),
        pltpu.VMEM((EB,), jnp.int32),
        pltpu.VMEM((EB,), jnp.int32),
        pltpu.VMEM((EB, H), jnp.float32),
        pltpu.VMEM((EB, H), jnp.float32),
        pltpu.VMEM((EB, H), jnp.float32),
        pltpu.VMEM((EB, H), jnp.float32),
        pltpu.VMEM((RCH, H), jnp.float32),
        pltpu.VMEM_SHARED((NP, H), jnp.float32),
        pltpu.VMEM_SHARED((NP, H), jnp.float32),
        pltpu.SemaphoreType.DMA,
    ],
)


def kernel(h, e, edge_index0, edge_index1, W, b):
    h = h.astype(jnp.float32)
    e = e.astype(jnp.float32)
    W = W.astype(jnp.float32)
    b = b.astype(jnp.float32)
    blocks = (edge_index0, edge_index1)
    for i in range(W.shape[0]):
        src = blocks[i][0].astype(jnp.int32)
        dst = blocks[i][1].astype(jnp.int32)
        wcat = jnp.concatenate([W[i, 0], W[i, 1], W[i, 3], W[i, 4]], axis=1)
        bcat = jnp.concatenate([b[i, 0], b[i, 1], b[i, 3], b[i, 4]])[None, :]
        ah, t = _node_mm(h, wcat, bcat)
        ce = _edge_mm(e, W[i, 2], b[i, 2][None, :])
        eflat, num, den = _edge_sc(
            t.reshape(2 * 3 * N_NODES, H), ce.reshape(NC * N_EDGES, H),
            src, dst)
        e = eflat.reshape(NC, N_EDGES, H)
        h = _node_upd(ah, num.reshape(NC, NP, H), den.reshape(NC, NP, H))
    return h, _merge_e(e)
